# baseline (speedup 1.0000x reference)
import jax
import jax.numpy as jnp
from jax import lax
from jax.experimental import pallas as pl
from jax.experimental.pallas import tpu as pltpu
from jax.experimental.pallas import tpu_sc as plsc

_N = 10000
_E = 320000
_D = 128
_EPS = 1e-5

_NS = 16
_NC = 2
_L = 16

_C3 = 160
_IB3 = 32
_EPT3 = _C3 * 128
_EP3 = _NS * _EPT3
_NP3 = 10112

_C1 = 79
_EPW1 = _C1 * 128
_EP1 = _NC * _NS * _EPW1
_NP1 = 10240

_B = 1000
_NB = _N // _B


def _mesh():
    return plsc.VectorSubcoreMesh(core_axis_name="c", subcore_axis_name="s")


def _deg_body(dstd_hbm, deg_hbm, idx_v, ones_v, zero_v, stage_v, acc_sh):
    cid = lax.axis_index("c")
    sid = lax.axis_index("s")
    for i in range(128 // _L):
        ones_v[pl.ds(i * _L, _L)] = jnp.ones((_L,), jnp.float32)
        zero_v[pl.ds(i * _L, _L)] = jnp.zeros((_L,), jnp.float32)
    base = sid * (_NP1 // _NS)
    for j in range(_NP1 // _NS // 128):
        pltpu.sync_copy(zero_v, acc_sh.at[pl.ds(base + j * 128, 128)])
    plsc.subcore_barrier()
    pltpu.sync_copy(dstd_hbm.at[cid, sid], idx_v)

    def step(j, _):
        pltpu.sync_copy(ones_v, acc_sh.at[idx_v.at[j]], add=True)
        return _

    lax.fori_loop(0, _C1, step, None)
    plsc.subcore_barrier()
    for j in range(_NP1 // _NS // 128):
        pltpu.sync_copy(acc_sh.at[pl.ds(base + j * 128, 128)], stage_v)
        pltpu.sync_copy(stage_v, deg_hbm.at[cid, pl.ds(base + j * 128, 128)])


def _deg_call(dstd):
    fn = pl.kernel(
        _deg_body,
        out_type=jax.ShapeDtypeStruct((_NC, _NP1), jnp.float32),
        mesh=_mesh(),
        scratch_types=[
            pltpu.VMEM((_C1, 128), jnp.int32),
            pltpu.VMEM((128,), jnp.float32),
            pltpu.VMEM((128,), jnp.float32),
            pltpu.VMEM((128,), jnp.float32),
            pltpu.VMEM_SHARED((_NP1,), jnp.float32),
        ],
    )
    return fn(dstd)


def _seg_body(table_hbm, src_hbm, dst_hbm, zblk_hbm, out_hbm,
              si0_v, di0_v, si1_v, di1_v, buf_a, buf_b, acc_sh,
              sem_a, sem_b, sem_i):
    cid = lax.axis_index("c")
    sid = lax.axis_index("s")
    pltpu.sync_copy(zblk_hbm, buf_a)
    zbase = sid * (_NP3 // _NS)
    for r0, rr in ((0, 128), (128, 128), (256, 128), (384, 128), (512, 120)):
        pltpu.sync_copy(buf_a.at[pl.ds(0, rr)],
                        acc_sh.at[pl.ds(zbase + r0, rr)])
    plsc.subcore_barrier()

    ibufs = ((si0_v, di0_v), (si1_v, di1_v))
    pltpu.sync_copy(src_hbm.at[cid, sid, pl.ds(0, _IB3)], si0_v)
    pltpu.sync_copy(dst_hbm.at[sid, pl.ds(0, _IB3)], di0_v)

    def gather(sv, j, buf, sem):
        pltpu.async_copy(table_hbm.at[sv.at[j, pl.ds(0, 64)]],
                         buf.at[pl.ds(0, 64)], sem)
        pltpu.async_copy(table_hbm.at[sv.at[j, pl.ds(64, 64)]],
                         buf.at[pl.ds(64, 64)], sem)

    def wait_scat(buf, sem, dv, j):
        pltpu.make_async_copy(table_hbm.at[dv.at[j]], buf, sem).wait()
        pltpu.sync_copy(buf, acc_sh.at[dv.at[j]], add=True)

    nblk = _C3 // _IB3
    gather(si0_v, 0, buf_a, sem_a)
    for b in range(nblk):
        si_v, di_v = ibufs[b % 2]
        if b + 1 < nblk:
            sn, dn = ibufs[(b + 1) % 2]
            pltpu.async_copy(src_hbm.at[cid, sid, pl.ds((b + 1) * _IB3, _IB3)],
                             sn, sem_i)
            pltpu.async_copy(dst_hbm.at[sid, pl.ds((b + 1) * _IB3, _IB3)],
                             dn, sem_i)

        def step(k, _):
            j = 2 * k
            gather(si_v, j + 1, buf_b, sem_b)
            wait_scat(buf_a, sem_a, di_v, j)
            gather(si_v, j + 2, buf_a, sem_a)
            wait_scat(buf_b, sem_b, di_v, j + 1)
            return _

        lax.fori_loop(0, _IB3 // 2 - 1, step, None)
        gather(si_v, _IB3 - 1, buf_b, sem_b)
        wait_scat(buf_a, sem_a, di_v, _IB3 - 2)
        if b + 1 < nblk:
            sn, dn = ibufs[(b + 1) % 2]
            pltpu.make_async_copy(
                src_hbm.at[cid, sid, pl.ds((b + 1) * _IB3, _IB3)],
                sn, sem_i).wait()
            pltpu.make_async_copy(
                dst_hbm.at[sid, pl.ds((b + 1) * _IB3, _IB3)],
                dn, sem_i).wait()
            gather(sn, 0, buf_a, sem_a)
        wait_scat(buf_b, sem_b, di_v, _IB3 - 1)
    plsc.subcore_barrier()

    for r0, rr in ((0, 128), (128, 128), (256, 128), (384, 128), (512, 120)):
        pltpu.sync_copy(acc_sh.at[pl.ds(zbase + r0, rr)],
                        buf_a.at[pl.ds(0, rr)])
        pltpu.sync_copy(buf_a.at[pl.ds(0, rr)],
                        out_hbm.at[cid, pl.ds(zbase + r0, rr)])


def _seg_call(table2, src2, dst3, zblk):
    fn = pl.kernel(
        _seg_body,
        out_type=jax.ShapeDtypeStruct((_NC, _NP3, _D), jnp.float32),
        mesh=_mesh(),
        scratch_types=[
            pltpu.VMEM((_IB3, 128), jnp.int32),
            pltpu.VMEM((_IB3, 128), jnp.int32),
            pltpu.VMEM((_IB3, 128), jnp.int32),
            pltpu.VMEM((_IB3, 128), jnp.int32),
            pltpu.VMEM((128, _D), jnp.float32),
            pltpu.VMEM((128, _D), jnp.float32),
            pltpu.VMEM_SHARED((_NP3, _D), jnp.float32),
            pltpu.SemaphoreType.DMA,
            pltpu.SemaphoreType.DMA,
            pltpu.SemaphoreType.DMA,
        ],
    )
    return fn(table2, src2, dst3, zblk)


def _prem_body(x_ref, wp_ref, bp_ref, hp_ref):
    hp_ref[...] = jnp.dot(x_ref[...], wp_ref[...],
                          preferred_element_type=jnp.float32) + bp_ref[...]


def _prem_call(x, W_pre, b_pre):
    return pl.pallas_call(
        _prem_body,
        grid=(_NB,),
        in_specs=[
            pl.BlockSpec((_B, _D), lambda i: (i, 0)),
            pl.BlockSpec((_D, _D), lambda i: (0, 0)),
            pl.BlockSpec((1, _D), lambda i: (0, 0)),
        ],
        out_specs=pl.BlockSpec((_B, _D), lambda i: (i, 0)),
        out_shape=jax.ShapeDtypeStruct((_N, _D), jnp.float32),
    )(x, W_pre, b_pre)


def _pre_body(hp_ref, dp_ref, tbl_ref, dis_ref, ideg_ref):
    deg = jnp.maximum(dp_ref[:, 0:1] + dp_ref[:, 1:2], 1.0)
    dis = lax.rsqrt(deg)
    hp = hp_ref[...]
    tbl_ref[0, :, :] = hp * dis
    tbl_ref[1, :, :] = hp
    dis_ref[...] = dis
    ideg_ref[...] = 1.0 / deg


def _pre_call(hp, dpT):
    return pl.pallas_call(
        _pre_body,
        grid=(_NB,),
        in_specs=[
            pl.BlockSpec((_B, _D), lambda i: (i, 0)),
            pl.BlockSpec((_B, 2), lambda i: (i, 0)),
        ],
        out_specs=[
            pl.BlockSpec((2, _B, _D), lambda i: (0, i, 0)),
            pl.BlockSpec((_B, 1), lambda i: (i, 0)),
            pl.BlockSpec((_B, 1), lambda i: (i, 0)),
        ],
        out_shape=[
            jax.ShapeDtypeStruct((2, _N, _D), jnp.float32),
            jax.ShapeDtypeStruct((_N, 1), jnp.float32),
            jax.ShapeDtypeStruct((_N, 1), jnp.float32),
        ],
    )(hp, dpT)


def _softmax2(a_ref):
    a = a_ref[...]
    m = jnp.max(a, axis=1, keepdims=True)
    e = jnp.exp(a - m)
    s = e / jnp.sum(e, axis=1, keepdims=True)
    return s[:, 0:1], s[:, 1:2]


def _mix_body(s_ref, tbl_ref, dis_ref, ideg_ref, gw_ref, gb_ref,
              ws_ref, wn_ref, sb_ref, ac_ref, t_ref, st_ref, acc):
    i = pl.program_id(0)
    a0, a1 = _softmax2(ac_ref)
    s1 = s_ref[0, :, :]
    s2 = s_ref[1, :, :]
    hp = tbl_ref[0, :, :]
    gcn = dis_ref[...] * jnp.dot(s1, gw_ref[...],
                                 preferred_element_type=jnp.float32) + gb_ref[...]
    sage = (jnp.dot(hp, ws_ref[...], preferred_element_type=jnp.float32)
            + jnp.dot(ideg_ref[...] * s2, wn_ref[...],
                      preferred_element_type=jnp.float32) + sb_ref[...])
    t = a0 * gcn + a1 * sage
    t_ref[...] = t

    @pl.when(i == 0)
    def _():
        acc[...] = jnp.zeros_like(acc)

    acc[0:1, :] += jnp.sum(t, axis=0, keepdims=True)
    acc[1:2, :] += jnp.sum(t * t, axis=0, keepdims=True)
    st_ref[...] = acc[...]


def _mix_call(S, table, dis, ideg, gcn_W, gcn_b, sage_W_self, sage_W_neigh,
              sage_b, a_conv):
    return pl.pallas_call(
        _mix_body,
        grid=(_NB,),
        in_specs=[
            pl.BlockSpec((2, _B, _D), lambda i: (0, i, 0)),
            pl.BlockSpec((1, _B, _D), lambda i: (1, i, 0)),
            pl.BlockSpec((_B, 1), lambda i: (i, 0)),
            pl.BlockSpec((_B, 1), lambda i: (i, 0)),
            pl.BlockSpec((_D, _D), lambda i: (0, 0)),
            pl.BlockSpec((1, _D), lambda i: (0, 0)),
            pl.BlockSpec((_D, _D), lambda i: (0, 0)),
            pl.BlockSpec((_D, _D), lambda i: (0, 0)),
            pl.BlockSpec((1, _D), lambda i: (0, 0)),
            pl.BlockSpec((1, 2), lambda i: (0, 0)),
        ],
        out_specs=[
            pl.BlockSpec((_B, _D), lambda i: (i, 0)),
            pl.BlockSpec((2, _D), lambda i: (0, 0)),
        ],
        out_shape=[
            jax.ShapeDtypeStruct((_N, _D), jnp.float32),
            jax.ShapeDtypeStruct((2, _D), jnp.float32),
        ],
        scratch_shapes=[pltpu.VMEM((2, _D), jnp.float32)],
    )(S, table, dis, ideg, gcn_W, gcn_b, sage_W_self, sage_W_neigh,
      sage_b, a_conv)


def _post_body(t_ref, st_ref, lg_ref, lb_ref, bg_ref, bb_ref, an_ref,
               aa_ref, wo_ref, bo_ref, o_ref):
    an0, an1 = _softmax2(an_ref)
    aa0, aa1 = _softmax2(aa_ref)
    t = t_ref[...]
    mu_b = st_ref[0:1, :] * (1.0 / _N)
    var_b = st_ref[1:2, :] * (1.0 / _N) - mu_b * mu_b
    bn = (t - mu_b) * lax.rsqrt(var_b + _EPS) * bg_ref[...] + bb_ref[...]
    mu = jnp.mean(t, axis=1, keepdims=True)
    var = jnp.mean(t * t, axis=1, keepdims=True) - mu * mu
    ln = (t - mu) * lax.rsqrt(var + _EPS) * lg_ref[...] + lb_ref[...]
    h = an0 * ln + an1 * bn
    r = h * aa0
    relu = jnp.maximum(r, 0.0)
    e = h * aa1
    elu = jnp.where(e > 0.0, e, jnp.exp(jnp.minimum(e, 0.0)) - 1.0)
    h2 = relu + elu
    o_ref[...] = jnp.dot(h2, wo_ref[...],
                         preferred_element_type=jnp.float32) + bo_ref[...]


def _post_call(t, stats, ln_gamma, ln_beta, bn_gamma, bn_beta, a_norm,
               a_act, W_post, b_post):
    return pl.pallas_call(
        _post_body,
        grid=(_NB,),
        in_specs=[
            pl.BlockSpec((_B, _D), lambda i: (i, 0)),
            pl.BlockSpec((2, _D), lambda i: (0, 0)),
            pl.BlockSpec((1, _D), lambda i: (0, 0)),
            pl.BlockSpec((1, _D), lambda i: (0, 0)),
            pl.BlockSpec((1, _D), lambda i: (0, 0)),
            pl.BlockSpec((1, _D), lambda i: (0, 0)),
            pl.BlockSpec((1, 2), lambda i: (0, 0)),
            pl.BlockSpec((1, 2), lambda i: (0, 0)),
            pl.BlockSpec((_D, _D), lambda i: (0, 0)),
            pl.BlockSpec((1, _D), lambda i: (0, 0)),
        ],
        out_specs=pl.BlockSpec((_B, _D), lambda i: (i, 0)),
        out_shape=jax.ShapeDtypeStruct((_N, _D), jnp.float32),
    )(t, stats, ln_gamma, ln_beta, bn_gamma, bn_beta, a_norm, a_act,
      W_post, b_post)


def kernel(x, edge_index, W_pre, b_pre, gcn_W, gcn_b, sage_W_self,
           sage_W_neigh, sage_b, ln_gamma, ln_beta, bn_gamma, bn_beta,
           alpha_conv, alpha_norm, alpha_act, W_post, b_post):
    src = edge_index[0]
    dst = edge_index[1]

    dstd = jnp.concatenate(
        [dst, jnp.full((_EP1 - _E,), _N, jnp.int32)]).reshape(_NC, _NS, _C1, 128)
    srcp = jnp.concatenate([src, jnp.zeros((_EP3 - _E,), jnp.int32)])
    dstp = jnp.concatenate([dst, jnp.full((_EP3 - _E,), _N, jnp.int32)])
    src2 = jnp.stack([srcp, srcp + _N]).reshape(_NC, _NS, _C3, 128)
    dst3 = dstp.reshape(_NS, _C3, 128)
    zblk = jnp.zeros((128, _D), jnp.float32)

    b_pre2 = b_pre.reshape(1, _D)
    gcn_b2 = gcn_b.reshape(1, _D)
    sage_b2 = sage_b.reshape(1, _D)
    ln_g2 = ln_gamma.reshape(1, _D)
    ln_b2 = ln_beta.reshape(1, _D)
    bn_g2 = bn_gamma.reshape(1, _D)
    bn_b2 = bn_beta.reshape(1, _D)
    b_post2 = b_post.reshape(1, _D)
    a_conv = alpha_conv.reshape(1, 2)
    a_norm = alpha_norm.reshape(1, 2)
    a_act = alpha_act.reshape(1, 2)

    degp = _deg_call(dstd)
    dpT = jnp.transpose(degp[:, :_N])

    hp = _prem_call(x, W_pre, b_pre2)
    table, dis, ideg = _pre_call(hp, dpT)
    table2 = table.reshape(2 * _N, _D)

    S = _seg_call(table2, src2, dst3, zblk)

    t, stats = _mix_call(S, table, dis, ideg, gcn_W, gcn_b2, sage_W_self,
                         sage_W_neigh, sage_b2, a_conv)

    return _post_call(t, stats, ln_g2, ln_b2, bn_g2, bn_b2, a_norm,
                      a_act, W_post, b_post2)


# speedup vs baseline: 1.1827x; 1.1827x over previous
import jax
import jax.numpy as jnp
from jax import lax
from jax.experimental import pallas as pl
from jax.experimental.pallas import tpu as pltpu
from jax.experimental.pallas import tpu_sc as plsc

_N = 10000
_E = 320000
_D = 128
_EPS = 1e-5

_NS = 16
_NC = 2
_L = 16

_C3 = 160
_IB3 = 32
_EPT3 = _C3 * 128
_EP3 = _NS * _EPT3
_NP3 = 10112

_C1 = 79
_EPW1 = _C1 * 128
_EP1 = _NC * _NS * _EPW1
_NP1 = 10240

_B = 1000
_NB = _N // _B


def _mesh():
    return plsc.VectorSubcoreMesh(core_axis_name="c", subcore_axis_name="s")


def _deg_body(dstd_hbm, deg_hbm, idx_v, ones_v, zero_v, stage_v, acc_sh):
    cid = lax.axis_index("c")
    sid = lax.axis_index("s")
    for i in range(128 // _L):
        ones_v[pl.ds(i * _L, _L)] = jnp.ones((_L,), jnp.float32)
        zero_v[pl.ds(i * _L, _L)] = jnp.zeros((_L,), jnp.float32)
    base = sid * (_NP1 // _NS)
    for j in range(_NP1 // _NS // 128):
        pltpu.sync_copy(zero_v, acc_sh.at[pl.ds(base + j * 128, 128)])
    plsc.subcore_barrier()
    pltpu.sync_copy(dstd_hbm.at[cid, sid], idx_v)

    def step(j, _):
        pltpu.sync_copy(ones_v, acc_sh.at[idx_v.at[j]], add=True)
        return _

    lax.fori_loop(0, _C1, step, None)
    plsc.subcore_barrier()
    for j in range(_NP1 // _NS // 128):
        pltpu.sync_copy(acc_sh.at[pl.ds(base + j * 128, 128)], stage_v)
        pltpu.sync_copy(stage_v, deg_hbm.at[cid, pl.ds(base + j * 128, 128)])


def _deg_call(dstd):
    fn = pl.kernel(
        _deg_body,
        out_type=jax.ShapeDtypeStruct((_NC, _NP1), jnp.float32),
        mesh=_mesh(),
        scratch_types=[
            pltpu.VMEM((_C1, 128), jnp.int32),
            pltpu.VMEM((128,), jnp.float32),
            pltpu.VMEM((128,), jnp.float32),
            pltpu.VMEM((128,), jnp.float32),
            pltpu.VMEM_SHARED((_NP1,), jnp.float32),
        ],
    )
    return fn(dstd)


def _seg_body(table_hbm, src_hbm, dst_hbm, zblk_hbm, out_hbm,
              si0_v, di0_v, si1_v, di1_v, buf_a, buf_b, acc_sh,
              sem_a, sem_b, sem_i):
    cid = lax.axis_index("c")
    sid = lax.axis_index("s")
    pltpu.sync_copy(zblk_hbm, buf_a)
    zbase = sid * (_NP3 // _NS)
    for r0, rr in ((0, 128), (128, 128), (256, 128), (384, 128), (512, 120)):
        pltpu.sync_copy(buf_a.at[pl.ds(0, rr)],
                        acc_sh.at[pl.ds(zbase + r0, rr)])
    plsc.subcore_barrier()

    ibufs = ((si0_v, di0_v), (si1_v, di1_v))
    pltpu.sync_copy(src_hbm.at[cid, sid, pl.ds(0, _IB3)], si0_v)
    pltpu.sync_copy(dst_hbm.at[sid, pl.ds(0, _IB3)], di0_v)

    def gather(sv, j, buf, sem):
        pltpu.async_copy(table_hbm.at[sv.at[j, pl.ds(0, 64)]],
                         buf.at[pl.ds(0, 64)], sem)
        pltpu.async_copy(table_hbm.at[sv.at[j, pl.ds(64, 64)]],
                         buf.at[pl.ds(64, 64)], sem)

    def wait_scat(buf, sem, dv, j):
        pltpu.make_async_copy(table_hbm.at[dv.at[j]], buf, sem).wait()
        pltpu.sync_copy(buf, acc_sh.at[dv.at[j]], add=True)

    nblk = _C3 // _IB3
    gather(si0_v, 0, buf_a, sem_a)
    for b in range(nblk):
        si_v, di_v = ibufs[b % 2]
        if b + 1 < nblk:
            sn, dn = ibufs[(b + 1) % 2]
            pltpu.async_copy(src_hbm.at[cid, sid, pl.ds((b + 1) * _IB3, _IB3)],
                             sn, sem_i)
            pltpu.async_copy(dst_hbm.at[sid, pl.ds((b + 1) * _IB3, _IB3)],
                             dn, sem_i)

        def step(k, _):
            j = 2 * k
            gather(si_v, j + 1, buf_b, sem_b)
            wait_scat(buf_a, sem_a, di_v, j)
            gather(si_v, j + 2, buf_a, sem_a)
            wait_scat(buf_b, sem_b, di_v, j + 1)
            return _

        lax.fori_loop(0, _IB3 // 2 - 1, step, None)
        gather(si_v, _IB3 - 1, buf_b, sem_b)
        wait_scat(buf_a, sem_a, di_v, _IB3 - 2)
        if b + 1 < nblk:
            sn, dn = ibufs[(b + 1) % 2]
            pltpu.make_async_copy(
                src_hbm.at[cid, sid, pl.ds((b + 1) * _IB3, _IB3)],
                sn, sem_i).wait()
            pltpu.make_async_copy(
                dst_hbm.at[sid, pl.ds((b + 1) * _IB3, _IB3)],
                dn, sem_i).wait()
            gather(sn, 0, buf_a, sem_a)
        wait_scat(buf_b, sem_b, di_v, _IB3 - 1)
    plsc.subcore_barrier()

    for r0, rr in ((0, 128), (128, 128), (256, 128), (384, 128), (512, 120)):
        pltpu.sync_copy(acc_sh.at[pl.ds(zbase + r0, rr)],
                        buf_a.at[pl.ds(0, rr)])
        pltpu.sync_copy(buf_a.at[pl.ds(0, rr)],
                        out_hbm.at[cid, pl.ds(zbase + r0, rr)])


def _seg_call(table2, src2, dst3, zblk):
    fn = pl.kernel(
        _seg_body,
        out_type=jax.ShapeDtypeStruct((_NC, _NP3, _D), jnp.float32),
        mesh=_mesh(),
        scratch_types=[
            pltpu.VMEM((_IB3, 128), jnp.int32),
            pltpu.VMEM((_IB3, 128), jnp.int32),
            pltpu.VMEM((_IB3, 128), jnp.int32),
            pltpu.VMEM((_IB3, 128), jnp.int32),
            pltpu.VMEM((128, _D), jnp.float32),
            pltpu.VMEM((128, _D), jnp.float32),
            pltpu.VMEM_SHARED((_NP3, _D), jnp.float32),
            pltpu.SemaphoreType.DMA,
            pltpu.SemaphoreType.DMA,
            pltpu.SemaphoreType.DMA,
        ],
    )
    return fn(table2, src2, dst3, zblk)


def _pre_body(x_ref, wp_ref, bp_ref, dp_ref, tbl_ref, dis_ref, ideg_ref):
    deg = jnp.maximum(dp_ref[:, 0:1] + dp_ref[:, 1:2], 1.0)
    dis = lax.rsqrt(deg)
    hp = jnp.dot(x_ref[...], wp_ref[...],
                 preferred_element_type=jnp.float32) + bp_ref[...]
    tbl_ref[0, :, :] = hp * dis
    tbl_ref[1, :, :] = hp
    dis_ref[...] = dis
    ideg_ref[...] = 1.0 / deg


def _pre_call(x, W_pre, b_pre, dpT):
    return pl.pallas_call(
        _pre_body,
        grid=(_NB,),
        in_specs=[
            pl.BlockSpec((_B, _D), lambda i: (i, 0)),
            pl.BlockSpec((_D, _D), lambda i: (0, 0)),
            pl.BlockSpec((1, _D), lambda i: (0, 0)),
            pl.BlockSpec((_B, 2), lambda i: (i, 0)),
        ],
        out_specs=[
            pl.BlockSpec((2, _B, _D), lambda i: (0, i, 0)),
            pl.BlockSpec((_B, 1), lambda i: (i, 0)),
            pl.BlockSpec((_B, 1), lambda i: (i, 0)),
        ],
        out_shape=[
            jax.ShapeDtypeStruct((2, _N, _D), jnp.float32),
            jax.ShapeDtypeStruct((_N, 1), jnp.float32),
            jax.ShapeDtypeStruct((_N, 1), jnp.float32),
        ],
    )(x, W_pre, b_pre, dpT)


def _softmax2(a_ref):
    a = a_ref[...]
    m = jnp.max(a, axis=1, keepdims=True)
    e = jnp.exp(a - m)
    s = e / jnp.sum(e, axis=1, keepdims=True)
    return s[:, 0:1], s[:, 1:2]


def _mix_body(s_ref, tbl_ref, dis_ref, ideg_ref, gw_ref, gb_ref,
              ws_ref, wn_ref, sb_ref, ac_ref, t_ref, st_ref, acc):
    i = pl.program_id(0)
    a0, a1 = _softmax2(ac_ref)
    s1 = s_ref[0, :, :]
    s2 = s_ref[1, :, :]
    hp = tbl_ref[0, :, :]
    gcn = dis_ref[...] * jnp.dot(s1, gw_ref[...],
                                 preferred_element_type=jnp.float32) + gb_ref[...]
    sage = (jnp.dot(hp, ws_ref[...], preferred_element_type=jnp.float32)
            + jnp.dot(ideg_ref[...] * s2, wn_ref[...],
                      preferred_element_type=jnp.float32) + sb_ref[...])
    t = a0 * gcn + a1 * sage
    t_ref[...] = t

    @pl.when(i == 0)
    def _():
        acc[...] = jnp.zeros_like(acc)

    acc[0:1, :] += jnp.sum(t, axis=0, keepdims=True)
    acc[1:2, :] += jnp.sum(t * t, axis=0, keepdims=True)
    st_ref[...] = acc[...]


def _mix_call(S, table, dis, ideg, gcn_W, gcn_b, sage_W_self, sage_W_neigh,
              sage_b, a_conv):
    return pl.pallas_call(
        _mix_body,
        grid=(_NB,),
        in_specs=[
            pl.BlockSpec((2, _B, _D), lambda i: (0, i, 0)),
            pl.BlockSpec((1, _B, _D), lambda i: (1, i, 0)),
            pl.BlockSpec((_B, 1), lambda i: (i, 0)),
            pl.BlockSpec((_B, 1), lambda i: (i, 0)),
            pl.BlockSpec((_D, _D), lambda i: (0, 0)),
            pl.BlockSpec((1, _D), lambda i: (0, 0)),
            pl.BlockSpec((_D, _D), lambda i: (0, 0)),
            pl.BlockSpec((_D, _D), lambda i: (0, 0)),
            pl.BlockSpec((1, _D), lambda i: (0, 0)),
            pl.BlockSpec((1, 2), lambda i: (0, 0)),
        ],
        out_specs=[
            pl.BlockSpec((_B, _D), lambda i: (i, 0)),
            pl.BlockSpec((2, _D), lambda i: (0, 0)),
        ],
        out_shape=[
            jax.ShapeDtypeStruct((_N, _D), jnp.float32),
            jax.ShapeDtypeStruct((2, _D), jnp.float32),
        ],
        scratch_shapes=[pltpu.VMEM((2, _D), jnp.float32)],
    )(S, table, dis, ideg, gcn_W, gcn_b, sage_W_self, sage_W_neigh,
      sage_b, a_conv)


def _post_body(t_ref, st_ref, lg_ref, lb_ref, bg_ref, bb_ref, an_ref,
               aa_ref, wo_ref, bo_ref, o_ref):
    an0, an1 = _softmax2(an_ref)
    aa0, aa1 = _softmax2(aa_ref)
    t = t_ref[...]
    mu_b = st_ref[0:1, :] * (1.0 / _N)
    var_b = st_ref[1:2, :] * (1.0 / _N) - mu_b * mu_b
    bn = (t - mu_b) * lax.rsqrt(var_b + _EPS) * bg_ref[...] + bb_ref[...]
    mu = jnp.mean(t, axis=1, keepdims=True)
    var = jnp.mean(t * t, axis=1, keepdims=True) - mu * mu
    ln = (t - mu) * lax.rsqrt(var + _EPS) * lg_ref[...] + lb_ref[...]
    h = an0 * ln + an1 * bn
    r = h * aa0
    relu = jnp.maximum(r, 0.0)
    e = h * aa1
    elu = jnp.where(e > 0.0, e, jnp.exp(jnp.minimum(e, 0.0)) - 1.0)
    h2 = relu + elu
    o_ref[...] = jnp.dot(h2, wo_ref[...],
                         preferred_element_type=jnp.float32) + bo_ref[...]


def _post_call(t, stats, ln_gamma, ln_beta, bn_gamma, bn_beta, a_norm,
               a_act, W_post, b_post):
    return pl.pallas_call(
        _post_body,
        grid=(_NB,),
        in_specs=[
            pl.BlockSpec((_B, _D), lambda i: (i, 0)),
            pl.BlockSpec((2, _D), lambda i: (0, 0)),
            pl.BlockSpec((1, _D), lambda i: (0, 0)),
            pl.BlockSpec((1, _D), lambda i: (0, 0)),
            pl.BlockSpec((1, _D), lambda i: (0, 0)),
            pl.BlockSpec((1, _D), lambda i: (0, 0)),
            pl.BlockSpec((1, 2), lambda i: (0, 0)),
            pl.BlockSpec((1, 2), lambda i: (0, 0)),
            pl.BlockSpec((_D, _D), lambda i: (0, 0)),
            pl.BlockSpec((1, _D), lambda i: (0, 0)),
        ],
        out_specs=pl.BlockSpec((_B, _D), lambda i: (i, 0)),
        out_shape=jax.ShapeDtypeStruct((_N, _D), jnp.float32),
    )(t, stats, ln_gamma, ln_beta, bn_gamma, bn_beta, a_norm, a_act,
      W_post, b_post)


def kernel(x, edge_index, W_pre, b_pre, gcn_W, gcn_b, sage_W_self,
           sage_W_neigh, sage_b, ln_gamma, ln_beta, bn_gamma, bn_beta,
           alpha_conv, alpha_norm, alpha_act, W_post, b_post):
    src = edge_index[0]
    dst = edge_index[1]

    dstd = jnp.concatenate(
        [dst, jnp.full((_EP1 - _E,), _N, jnp.int32)]).reshape(_NC, _NS, _C1, 128)
    srcp = jnp.concatenate([src, jnp.zeros((_EP3 - _E,), jnp.int32)])
    dstp = jnp.concatenate([dst, jnp.full((_EP3 - _E,), _N, jnp.int32)])
    src2 = jnp.stack([srcp, srcp + _N]).reshape(_NC, _NS, _C3, 128)
    dst3 = dstp.reshape(_NS, _C3, 128)
    zblk = jnp.zeros((128, _D), jnp.float32)

    b_pre2 = b_pre.reshape(1, _D)
    gcn_b2 = gcn_b.reshape(1, _D)
    sage_b2 = sage_b.reshape(1, _D)
    ln_g2 = ln_gamma.reshape(1, _D)
    ln_b2 = ln_beta.reshape(1, _D)
    bn_g2 = bn_gamma.reshape(1, _D)
    bn_b2 = bn_beta.reshape(1, _D)
    b_post2 = b_post.reshape(1, _D)
    a_conv = alpha_conv.reshape(1, 2)
    a_norm = alpha_norm.reshape(1, 2)
    a_act = alpha_act.reshape(1, 2)

    degp = _deg_call(dstd)
    dpT = jnp.transpose(degp[:, :_N])

    table, dis, ideg = _pre_call(x, W_pre, b_pre2, dpT)
    table2 = table.reshape(2 * _N, _D)

    S = _seg_call(table2, src2, dst3, zblk)

    t, stats = _mix_call(S, table, dis, ideg, gcn_W, gcn_b2, sage_W_self,
                         sage_W_neigh, sage_b2, a_conv)

    return _post_call(t, stats, ln_g2, ln_b2, bn_g2, bn_b2, a_norm,
                      a_act, W_post, b_post2)
